# Initial kernel scaffold; baseline (speedup 1.0000x reference)
import jax
import jax.numpy as jnp
from jax.experimental import pallas as pl


def kernel(z, edge_label_index):
    raise NotImplementedError("write your pallas kernel here")


# speedup vs baseline: 4.8384x; 4.8384x over previous
import functools

import jax
import jax.numpy as jnp
from jax import lax
from jax.experimental import pallas as pl
from jax.experimental.pallas import tpu as pltpu
from jax.experimental.pallas import tpu_sc as plsc

N_NODES = 10000
N_FEAT = 128
N_EDGES = 320000

_INFO = plsc.get_sparse_core_info()
NC, NS = _INFO.num_cores, _INFO.num_subcores
NW = NC * NS
PER_W = N_EDGES // NW
C = 400
NCHUNK = PER_W // C


def _decode_body(z_hbm, elix_hbm, out_hbm, sidx_v, didx_v, srows_v, drows_v,
                 part_v, outc_v, sem_s, sem_d):
    wid = lax.axis_index("s") * NC + lax.axis_index("c")
    lane16 = lax.iota(jnp.int32, 16) * 16

    def chunk_body(ci, carry):
        base = wid * PER_W + ci * C
        pltpu.sync_copy(elix_hbm.at[pl.ds(base, C)], sidx_v)
        pltpu.sync_copy(elix_hbm.at[pl.ds(N_EDGES + base, C)], didx_v)
        cp_s = pltpu.async_copy(z_hbm.at[sidx_v], srows_v, sem_s)
        cp_d = pltpu.async_copy(z_hbm.at[didx_v], drows_v, sem_d)
        cp_s.wait()
        cp_d.wait()

        def edge_body(e, c2):
            acc = srows_v[e, pl.ds(0, 16)] * drows_v[e, pl.ds(0, 16)]
            for k in range(1, 8):
                acc = acc + (srows_v[e, pl.ds(k * 16, 16)] *
                             drows_v[e, pl.ds(k * 16, 16)])
            part_v[pl.ds(e * 16, 16)] = acc
            return c2

        lax.fori_loop(0, C, edge_body, 0, unroll=4)

        def grp_body(g, c2):
            gbase = g * 256
            t = plsc.load_gather(part_v, [lane16 + gbase])
            for j in range(1, 16):
                t = t + plsc.load_gather(part_v, [lane16 + (gbase + j)])
            outc_v[pl.ds(g * 16, 16)] = t
            return c2

        lax.fori_loop(0, C // 16, grp_body, 0)
        pltpu.sync_copy(outc_v, out_hbm.at[pl.ds(base, C)])
        return carry

    lax.fori_loop(0, NCHUNK, chunk_body, 0)


@functools.partial(jax.jit, static_argnames=())
def kernel(z, edge_label_index):
    elix = edge_label_index.astype(jnp.int32).reshape(2 * N_EDGES)
    mesh = plsc.VectorSubcoreMesh(core_axis_name="c", subcore_axis_name="s")
    k = pl.kernel(
        _decode_body,
        mesh=mesh,
        compiler_params=pltpu.CompilerParams(needs_layout_passes=False),
        out_type=jax.ShapeDtypeStruct((N_EDGES,), jnp.float32),
        scratch_types=[
            pltpu.VMEM((C,), jnp.int32),
            pltpu.VMEM((C,), jnp.int32),
            pltpu.VMEM((C, N_FEAT), jnp.float32),
            pltpu.VMEM((C, N_FEAT), jnp.float32),
            pltpu.VMEM((C * 16,), jnp.float32),
            pltpu.VMEM((C,), jnp.float32),
            pltpu.SemaphoreType.DMA,
            pltpu.SemaphoreType.DMA,
        ],
    )
    return k(z, elix)


# speedup vs baseline: 8.3582x; 1.7274x over previous
import jax
import jax.numpy as jnp
from jax import lax
from jax.experimental import pallas as pl
from jax.experimental.pallas import tpu as pltpu
from jax.experimental.pallas import tpu_sc as plsc

N_NODES = 10000
N_FEAT = 128
N_EDGES = 320000

_INFO = plsc.get_sparse_core_info()
NC, NS = _INFO.num_cores, _INFO.num_subcores
NW = NC * NS
PER_W = N_EDGES // NW
C = 200
NCHUNK = PER_W // C
NGRP = (C + 15) // 16
CP = NGRP * 16


def _decode_body(z_hbm, elix_hbm, out_hbm, sidx, didx, rs0, rs1, rd0, rd1,
                 part, ob0, ob1, ss0, ss1, sd0, sd1, os0, os1):
    wid = lax.axis_index("s") * NC + lax.axis_index("c")
    woff = wid * PER_W
    lane16 = lax.iota(jnp.int32, 16) * 16
    rs, rd, ob = (rs0, rs1), (rd0, rd1), (ob0, ob1)
    ssem, dsem, osem = (ss0, ss1), (sd0, sd1), (os0, os1)

    pltpu.sync_copy(elix_hbm.at[pl.ds(woff, PER_W)], sidx)
    pltpu.sync_copy(elix_hbm.at[pl.ds(N_EDGES + woff, PER_W)], didx)

    def start_gather(c, b):
        pltpu.async_copy(z_hbm.at[sidx.at[pl.ds(c * C, C)]], rs[b], ssem[b])
        pltpu.async_copy(z_hbm.at[didx.at[pl.ds(c * C, C)]], rd[b], dsem[b])

    def wait_gather(b):
        pltpu.make_async_copy(z_hbm.at[pl.ds(0, C)], rs[b], ssem[b]).wait()
        pltpu.make_async_copy(z_hbm.at[pl.ds(0, C)], rd[b], dsem[b]).wait()

    def compute(c, b):
        srows, drows = rs[b], rd[b]

        def edge_body(e, c2):
            acc = srows[e, pl.ds(0, 16)] * drows[e, pl.ds(0, 16)]
            for k in range(1, 8):
                acc = acc + (srows[e, pl.ds(k * 16, 16)] *
                             drows[e, pl.ds(k * 16, 16)])
            part[pl.ds(e * 16, 16)] = acc
            return c2

        lax.fori_loop(0, C, edge_body, 0, unroll=4)

        @pl.when(c >= 2)
        def _():
            pltpu.make_async_copy(ob[b].at[pl.ds(0, C)],
                                  out_hbm.at[pl.ds(0, C)], osem[b]).wait()

        def grp_body(g, c2):
            gbase = g * 256
            t = plsc.load_gather(part, [lane16 + gbase])
            for j in range(1, 16):
                t = t + plsc.load_gather(part, [lane16 + (gbase + j)])
            ob[b][pl.ds(g * 16, 16)] = t
            return c2

        lax.fori_loop(0, NGRP, grp_body, 0)
        pltpu.async_copy(ob[b].at[pl.ds(0, C)],
                         out_hbm.at[pl.ds(woff + c * C, C)], osem[b])

    start_gather(0, 0)

    def outer(ci2, carry):
        c0 = ci2 * 2
        start_gather(c0 + 1, 1)
        wait_gather(0)
        compute(c0, 0)

        @pl.when(c0 + 2 < NCHUNK)
        def _():
            start_gather(c0 + 2, 0)

        wait_gather(1)
        compute(c0 + 1, 1)
        return carry

    lax.fori_loop(0, NCHUNK // 2, outer, 0)
    pltpu.make_async_copy(ob[0].at[pl.ds(0, C)], out_hbm.at[pl.ds(0, C)],
                          osem[0]).wait()
    pltpu.make_async_copy(ob[1].at[pl.ds(0, C)], out_hbm.at[pl.ds(0, C)],
                          osem[1]).wait()


def kernel(z, edge_label_index):
    elix = edge_label_index.astype(jnp.int32).reshape(2 * N_EDGES)
    mesh = plsc.VectorSubcoreMesh(core_axis_name="c", subcore_axis_name="s")
    k = pl.kernel(
        _decode_body,
        mesh=mesh,
        compiler_params=pltpu.CompilerParams(needs_layout_passes=False),
        out_type=jax.ShapeDtypeStruct((N_EDGES,), jnp.float32),
        scratch_types=[
            pltpu.VMEM((PER_W,), jnp.int32),
            pltpu.VMEM((PER_W,), jnp.int32),
            pltpu.VMEM((C, N_FEAT), jnp.float32),
            pltpu.VMEM((C, N_FEAT), jnp.float32),
            pltpu.VMEM((C, N_FEAT), jnp.float32),
            pltpu.VMEM((C, N_FEAT), jnp.float32),
            pltpu.VMEM((CP * 16,), jnp.float32),
            pltpu.VMEM((CP,), jnp.float32),
            pltpu.VMEM((CP,), jnp.float32),
            pltpu.SemaphoreType.DMA,
            pltpu.SemaphoreType.DMA,
            pltpu.SemaphoreType.DMA,
            pltpu.SemaphoreType.DMA,
            pltpu.SemaphoreType.DMA,
            pltpu.SemaphoreType.DMA,
        ],
    )
    return k(z, elix)
